# baseline (speedup 1.0000x reference)
import functools

import numpy as np

import jax
import jax.numpy as jnp
from jax import lax
from jax.experimental import pallas as pl
from jax.experimental.pallas import tpu as pltpu
from jax.experimental.pallas import tpu_sc as plsc

BATCH = 16384
D = 128
LANES = 16
GROUPS = D // (2 * LANES)

_info = plsc.get_sparse_core_info()
NC, NS = _info.num_cores, _info.num_subcores
NW = NC * NS
ROWS_W = BATCH // NW
CHUNK = 128
NCHUNK = ROWS_W // CHUNK

_PERM = np.concatenate(
    [np.concatenate([np.arange(g * 32, (g + 1) * 32, 2),
                     np.arange(g * 32 + 1, (g + 1) * 32, 2)])
     for g in range(GROUPS)])


def _make_sc_call():
    mesh = plsc.VectorSubcoreMesh(core_axis_name="c", subcore_axis_name="s")

    @functools.partial(
        pl.kernel,
        mesh=mesh,
        out_type=jax.ShapeDtypeStruct((NW, LANES), jnp.float32),
        scratch_types=[
            pltpu.VMEM((ROWS_W,), jnp.int32),
            pltpu.VMEM((2, CHUNK, D), jnp.float32),
            pltpu.VMEM((2, CHUNK * D // 2), jnp.float32),
            pltpu.VMEM((LANES,), jnp.float32),
            pltpu.SemaphoreType.DMA,
            pltpu.SemaphoreType.DMA,
            pltpu.SemaphoreType.DMA,
            pltpu.SemaphoreType.DMA,
        ],
    )
    def sc_center_loss(feat_hbm, label_hbm, centers_hbm, out_hbm,
                       idx_v, cent_v, feat_v, out_v,
                       gsem0, gsem1, fsem0, fsem1):
        wid = lax.axis_index("s") * NC + lax.axis_index("c")
        base = wid * ROWS_W
        gsems = (gsem0, gsem1)
        fsems = (fsem0, fsem1)

        pltpu.sync_copy(label_hbm.at[pl.ds(base, ROWS_W)], idx_v)

        def start(c, slot):
            g = pltpu.async_copy(centers_hbm.at[idx_v.at[pl.ds(c * CHUNK, CHUNK)]],
                                 cent_v.at[slot], gsems[slot])
            flat_off = pl.multiple_of((base + c * CHUNK) * (D // 2),
                                      CHUNK * D // 2)
            f = pltpu.async_copy(
                feat_hbm.at[pl.ds(flat_off, CHUNK * D // 2)],
                feat_v.at[slot], fsems[slot])
            return g, f

        def compute(slot, accs):
            fv = feat_v.at[slot]
            cv = cent_v.at[slot]

            def body(i, accs):
                off = pl.multiple_of(i * (D // 2), D // 2)
                new = list(accs)
                for g in range(GROUPS):
                    w = jax.lax.bitcast_convert_type(
                        fv[pl.ds(off + g * LANES, LANES)], jnp.int32)
                    flo = jax.lax.bitcast_convert_type(w << 16, jnp.float32)
                    fhi = jax.lax.bitcast_convert_type(w & jnp.int32(-65536),
                                                       jnp.float32)
                    clo = cv[i, pl.ds(g * 2 * LANES, LANES)]
                    chi = cv[i, pl.ds(g * 2 * LANES + LANES, LANES)]
                    dlo = flo - clo
                    dhi = fhi - chi
                    new[2 * g] = new[2 * g] + dlo * dlo
                    new[2 * g + 1] = new[2 * g + 1] + dhi * dhi
                return tuple(new)

            return lax.fori_loop(0, CHUNK, body, accs)

        accs = tuple(jnp.zeros((LANES,), jnp.float32) for _ in range(2 * GROUPS))
        copies = {0: start(0, 0)}
        for c in range(NCHUNK):
            if c + 1 < NCHUNK:
                copies[c + 1] = start(c + 1, (c + 1) % 2)
            g, f = copies.pop(c)
            g.wait()
            f.wait()
            accs = compute(c % 2, accs)

        total = accs[0]
        for j in range(1, 2 * GROUPS):
            total = total + accs[j]
        out_v[...] = total * 0.5
        pltpu.sync_copy(out_v, out_hbm.at[wid])

    return sc_center_loss


_sc_center_loss = _make_sc_call()


def kernel(feat, label, centers):
    f_pairs = jax.lax.bitcast_convert_type(
        feat.astype(jnp.bfloat16).reshape(BATCH * D // 2, 2), jnp.float32)
    c_perm = centers[:, _PERM]
    partials = _sc_center_loss(f_pairs, label.astype(jnp.int32), c_perm)
    return jnp.sum(partials)


# speedup vs baseline: 21.0434x; 21.0434x over previous
import functools

import jax
import jax.numpy as jnp
from jax import lax
from jax.experimental import pallas as pl
from jax.experimental.pallas import tpu as pltpu
from jax.experimental.pallas import tpu_sc as plsc

BATCH = 16384
D = 128
LANES = 16
GROUPS = D // (2 * LANES)

_info = plsc.get_sparse_core_info()
NC, NS = _info.num_cores, _info.num_subcores
NW = NC * NS
ROWS_W = BATCH // NW
CHUNK = 128
NCHUNK = ROWS_W // CHUNK


def _make_sc_call():
    mesh = plsc.VectorSubcoreMesh(core_axis_name="c", subcore_axis_name="s")

    @functools.partial(
        pl.kernel,
        mesh=mesh,
        out_type=jax.ShapeDtypeStruct((NW, LANES), jnp.float32),
        scratch_types=[
            pltpu.VMEM((ROWS_W + LANES,), jnp.int32),
            pltpu.VMEM((ROWS_W,), jnp.int32),
            pltpu.VMEM((2, CHUNK, D), jnp.float32),
            pltpu.VMEM((2, CHUNK, D), jnp.float32),
            pltpu.VMEM((LANES,), jnp.float32),
            pltpu.SemaphoreType.DMA,
            pltpu.SemaphoreType.DMA,
            pltpu.SemaphoreType.DMA,
            pltpu.SemaphoreType.DMA,
        ],
    )
    def sc_center_loss(feat_hbm, label_hbm, ctable_hbm, out_hbm,
                       idx_v, idx2_v, cent_v, feat_v, out_v,
                       gsem0, gsem1, fsem0, fsem1):
        wid = lax.axis_index("s") * NC + lax.axis_index("c")
        base = wid * ROWS_W
        gsems = (gsem0, gsem1)
        fsems = (fsem0, fsem1)

        pltpu.sync_copy(label_hbm.at[pl.ds(base, ROWS_W)],
                        idx_v.at[pl.ds(0, ROWS_W)])
        for k in range(ROWS_W // LANES):
            idx2_v[pl.ds(k * LANES, LANES)] = (
                idx_v[pl.ds(k * LANES, LANES)] >> 1)

        def start(c, slot):
            g = pltpu.async_copy(ctable_hbm.at[idx2_v.at[pl.ds(c * CHUNK, CHUNK)]],
                                 cent_v.at[slot], gsems[slot])
            f = pltpu.async_copy(feat_hbm.at[pl.ds(base + c * CHUNK, CHUNK)],
                                 feat_v.at[slot], fsems[slot])
            return g, f

        def compute(c, slot, accs):
            fv = feat_v.at[slot]
            cv = cent_v.at[slot]

            def body(i, accs):
                lab = idx_v[pl.ds(c * CHUNK + i, LANES)][0]
                half = pl.multiple_of((lab & 1) * (D // 2), D // 2)
                new = list(accs)
                for g in range(GROUPS):
                    w = jax.lax.bitcast_convert_type(
                        cv[i, pl.ds(half + g * LANES, LANES)], jnp.int32)
                    clo = jax.lax.bitcast_convert_type(w << 16, jnp.float32)
                    chi = jax.lax.bitcast_convert_type(w & jnp.int32(-65536),
                                                       jnp.float32)
                    flo = fv[i, pl.ds(g * 2 * LANES, LANES)]
                    fhi = fv[i, pl.ds(g * 2 * LANES + LANES, LANES)]
                    dlo = flo - clo
                    dhi = fhi - chi
                    new[2 * g] = new[2 * g] + dlo * dlo
                    new[2 * g + 1] = new[2 * g + 1] + dhi * dhi
                return tuple(new)

            return lax.fori_loop(0, CHUNK, body, accs)

        accs = tuple(jnp.zeros((LANES,), jnp.float32) for _ in range(2 * GROUPS))
        copies = {0: start(0, 0)}
        for c in range(NCHUNK):
            if c + 1 < NCHUNK:
                copies[c + 1] = start(c + 1, (c + 1) % 2)
            g, f = copies.pop(c)
            g.wait()
            f.wait()
            accs = compute(c, c % 2, accs)

        total = accs[0]
        for j in range(1, 2 * GROUPS):
            total = total + accs[j]
        out_v[...] = total * 0.5
        pltpu.sync_copy(out_v, out_hbm.at[wid])

    return sc_center_loss


_sc_center_loss = _make_sc_call()


def _pack_centers(centers):
    n = centers.shape[0]
    bits = jax.lax.bitcast_convert_type(centers, jnp.int32)
    rne = (bits + jnp.int32(0x7FFF) + ((bits >> 16) & 1)) >> 16
    rne = rne & jnp.int32(0xFFFF)
    quads = rne.reshape(n, GROUPS, 2, LANES)
    words = quads[:, :, 0, :] | (quads[:, :, 1, :] << 16)
    packed = words.reshape(n // 2, D)
    return jax.lax.bitcast_convert_type(packed, jnp.float32)


def kernel(feat, label, centers):
    partials = _sc_center_loss(feat, label.astype(jnp.int32),
                               _pack_centers(centers))
    return jnp.sum(partials)


# speedup vs baseline: 23.6837x; 1.1255x over previous
import functools

import jax
import jax.numpy as jnp
from jax import lax
from jax.experimental import pallas as pl
from jax.experimental.pallas import tpu as pltpu
from jax.experimental.pallas import tpu_sc as plsc

BATCH = 16384
D = 128
LANES = 16
GROUPS = D // (2 * LANES)

_info = plsc.get_sparse_core_info()
NC, NS = _info.num_cores, _info.num_subcores
NW = NC * NS
ROWS_W = BATCH // NW
CHUNK = 128
NCHUNK = ROWS_W // CHUNK


def _make_sc_call():
    mesh = plsc.VectorSubcoreMesh(core_axis_name="c", subcore_axis_name="s")

    @functools.partial(
        pl.kernel,
        mesh=mesh,
        out_type=jax.ShapeDtypeStruct((NW, LANES), jnp.float32),
        scratch_types=[
            pltpu.VMEM((ROWS_W,), jnp.int32),
            pltpu.VMEM((2, CHUNK, D), jnp.float32),
            pltpu.VMEM((2, CHUNK, D), jnp.float32),
            pltpu.VMEM((LANES,), jnp.float32),
            pltpu.SemaphoreType.DMA,
            pltpu.SemaphoreType.DMA,
            pltpu.SemaphoreType.DMA,
            pltpu.SemaphoreType.DMA,
        ],
    )
    def sc_center_loss(feat_hbm, label_hbm, ctable_hbm, out_hbm,
                       idx_v, cent_v, feat_v, out_v,
                       gsem0, gsem1, fsem0, fsem1):
        wid = lax.axis_index("s") * NC + lax.axis_index("c")
        base = wid * ROWS_W
        gsems = (gsem0, gsem1)
        fsems = (fsem0, fsem1)

        pltpu.sync_copy(label_hbm.at[pl.ds(base, ROWS_W)], idx_v)

        def start(c, slot):
            g = pltpu.async_copy(ctable_hbm.at[idx_v.at[pl.ds(c * CHUNK, CHUNK)]],
                                 cent_v.at[slot], gsems[slot])
            f = pltpu.async_copy(feat_hbm.at[pl.ds(base + c * CHUNK, CHUNK)],
                                 feat_v.at[slot], fsems[slot])
            return g, f

        def compute(c, slot, accs):
            fv = feat_v.at[slot]
            cv = cent_v.at[slot]

            def body(i, accs):
                new = list(accs)
                for g in range(GROUPS):
                    w = jax.lax.bitcast_convert_type(
                        cv[i, pl.ds(g * LANES, LANES)], jnp.int32)
                    clo = jax.lax.bitcast_convert_type(w << 16, jnp.float32)
                    chi = jax.lax.bitcast_convert_type(w & jnp.int32(-65536),
                                                       jnp.float32)
                    flo = fv[i, pl.ds(g * 2 * LANES, LANES)]
                    fhi = fv[i, pl.ds(g * 2 * LANES + LANES, LANES)]
                    dlo = flo - clo
                    dhi = fhi - chi
                    new[2 * g] = new[2 * g] + dlo * dlo
                    new[2 * g + 1] = new[2 * g + 1] + dhi * dhi
                return tuple(new)

            return lax.fori_loop(0, CHUNK, body, accs)

        accs = tuple(jnp.zeros((LANES,), jnp.float32) for _ in range(2 * GROUPS))
        copies = {0: start(0, 0)}
        for c in range(NCHUNK):
            if c + 1 < NCHUNK:
                copies[c + 1] = start(c + 1, (c + 1) % 2)
            g, f = copies.pop(c)
            g.wait()
            f.wait()
            accs = compute(c, c % 2, accs)

        total = accs[0]
        for j in range(1, 2 * GROUPS):
            total = total + accs[j]
        out_v[...] = total * 0.5
        pltpu.sync_copy(out_v, out_hbm.at[wid])

    return sc_center_loss


_sc_center_loss = _make_sc_call()


def _pack_centers(centers):
    n = centers.shape[0]
    bits = jax.lax.bitcast_convert_type(centers, jnp.int32)
    rne = (bits + jnp.int32(0x7FFF) + ((bits >> 16) & 1)) >> 16
    rne = rne & jnp.int32(0xFFFF)
    quads = rne.reshape(n, GROUPS, 2, LANES)
    words = quads[:, :, 0, :] | (quads[:, :, 1, :] << 16)
    packed = jnp.pad(words.reshape(n, D // 2), ((0, 0), (0, D // 2)))
    return jax.lax.bitcast_convert_type(packed, jnp.float32)


def kernel(feat, label, centers):
    partials = _sc_center_loss(feat, label.astype(jnp.int32),
                               _pack_centers(centers))
    return jnp.sum(partials)


# speedup vs baseline: 23.7023x; 1.0008x over previous
import functools

import jax
import jax.numpy as jnp
from jax import lax
from jax.experimental import pallas as pl
from jax.experimental.pallas import tpu as pltpu
from jax.experimental.pallas import tpu_sc as plsc

BATCH = 16384
D = 128
LANES = 16
GROUPS = D // (2 * LANES)

_info = plsc.get_sparse_core_info()
NC, NS = _info.num_cores, _info.num_subcores
NW = NC * NS
ROWS_W = BATCH // NW
CHUNK = 128
NCHUNK = ROWS_W // CHUNK


def _make_sc_call():
    mesh = plsc.VectorSubcoreMesh(core_axis_name="c", subcore_axis_name="s")

    @functools.partial(
        pl.kernel,
        mesh=mesh,
        out_type=jax.ShapeDtypeStruct((NW, LANES), jnp.float32),
        scratch_types=[
            pltpu.VMEM((ROWS_W,), jnp.int32),
            pltpu.VMEM((2, CHUNK, D), jnp.float32),
            pltpu.VMEM((NCHUNK, CHUNK, D), jnp.float32),
            pltpu.VMEM((LANES,), jnp.float32),
            pltpu.SemaphoreType.DMA,
            pltpu.SemaphoreType.DMA,
            pltpu.SemaphoreType.DMA,
            pltpu.SemaphoreType.DMA,
            pltpu.SemaphoreType.DMA,
            pltpu.SemaphoreType.DMA,
        ],
    )
    def sc_center_loss(feat_hbm, label_hbm, ctable_hbm, out_hbm,
                       idx_v, cent_v, feat_v, out_v,
                       gsem0, gsem1, fsem0, fsem1, fsem2, fsem3):
        wid = lax.axis_index("s") * NC + lax.axis_index("c")
        base = wid * ROWS_W
        gsems = (gsem0, gsem1)
        fsems = (fsem0, fsem1, fsem2, fsem3)

        fcopies = [
            pltpu.async_copy(feat_hbm.at[pl.ds(base + c * CHUNK, CHUNK)],
                             feat_v.at[c], fsems[c])
            for c in range(NCHUNK)
        ]
        pltpu.sync_copy(label_hbm.at[pl.ds(base, ROWS_W)], idx_v)

        def start(c, slot):
            g = pltpu.async_copy(ctable_hbm.at[idx_v.at[pl.ds(c * CHUNK, CHUNK)]],
                                 cent_v.at[slot], gsems[slot])
            return g, fcopies[c]

        def compute(c, slot, accs):
            fv = feat_v.at[c]
            cv = cent_v.at[slot]

            def body(i, accs):
                new = list(accs)
                for g in range(GROUPS):
                    w = jax.lax.bitcast_convert_type(
                        cv[i, pl.ds(g * LANES, LANES)], jnp.int32)
                    clo = jax.lax.bitcast_convert_type(w << 16, jnp.float32)
                    chi = jax.lax.bitcast_convert_type(w & jnp.int32(-65536),
                                                       jnp.float32)
                    flo = fv[i, pl.ds(g * 2 * LANES, LANES)]
                    fhi = fv[i, pl.ds(g * 2 * LANES + LANES, LANES)]
                    dlo = flo - clo
                    dhi = fhi - chi
                    new[2 * g] = new[2 * g] + dlo * dlo
                    new[2 * g + 1] = new[2 * g + 1] + dhi * dhi
                return tuple(new)

            return lax.fori_loop(0, CHUNK, body, accs)

        accs = tuple(jnp.zeros((LANES,), jnp.float32) for _ in range(2 * GROUPS))
        copies = {0: start(0, 0)}
        for c in range(NCHUNK):
            if c + 1 < NCHUNK:
                copies[c + 1] = start(c + 1, (c + 1) % 2)
            g, f = copies.pop(c)
            g.wait()
            f.wait()
            accs = compute(c, c % 2, accs)

        total = accs[0]
        for j in range(1, 2 * GROUPS):
            total = total + accs[j]
        out_v[...] = total * 0.5
        pltpu.sync_copy(out_v, out_hbm.at[wid])

    return sc_center_loss


_sc_center_loss = _make_sc_call()


def _pack_centers(centers):
    n = centers.shape[0]
    bits = jax.lax.bitcast_convert_type(centers, jnp.int32)
    rne = (bits + jnp.int32(0x7FFF) + ((bits >> 16) & 1)) >> 16
    rne = rne & jnp.int32(0xFFFF)
    quads = rne.reshape(n, GROUPS, 2, LANES)
    words = quads[:, :, 0, :] | (quads[:, :, 1, :] << 16)
    packed = jnp.pad(words.reshape(n, D // 2), ((0, 0), (0, D // 2)))
    return jax.lax.bitcast_convert_type(packed, jnp.float32)


def kernel(feat, label, centers):
    partials = _sc_center_loss(feat, label.astype(jnp.int32),
                               _pack_centers(centers))
    return jnp.sum(partials)
